# Initial kernel scaffold; baseline (speedup 1.0000x reference)
import jax
import jax.numpy as jnp
from jax.experimental import pallas as pl


def kernel(fids_batch, table):
    raise NotImplementedError("write your pallas kernel here")


# speedup vs baseline: 1.5732x; 1.5732x over previous
import functools

import jax
import jax.numpy as jnp
from jax import lax
from jax.experimental import pallas as pl
from jax.experimental.pallas import tpu as pltpu
from jax.experimental.pallas import tpu_sc as plsc


@functools.cache
def _build(B, F):
    info = plsc.get_sparse_core_info()
    NW = info.num_cores * info.num_subcores
    L = info.num_lanes
    b_per_w = B // NW
    n_idx = b_per_w * F

    mesh = plsc.VectorSubcoreMesh(core_axis_name="c", subcore_axis_name="s")

    @functools.partial(
        pl.kernel,
        out_type=jax.ShapeDtypeStruct((B,), jnp.float32),
        mesh=mesh,
        scratch_types=[
            pltpu.VMEM((n_idx,), jnp.int32),
            pltpu.VMEM((n_idx,), jnp.float32),
            pltpu.VMEM((b_per_w,), jnp.float32),
            pltpu.SemaphoreType.DMA,
        ],
        compiler_params=pltpu.CompilerParams(needs_layout_passes=False),
    )
    def lr_pool(fids_hbm, table_hbm, out_hbm, idx_v, vals_v, out_v, sem):
        wid = lax.axis_index("s") * info.num_cores + lax.axis_index("c")
        base = wid * b_per_w
        pltpu.sync_copy(fids_hbm.at[pl.ds(base * F, n_idx)], idx_v)
        pltpu.async_copy(table_hbm.at[idx_v], vals_v, sem).wait()

        lanes = lax.iota(jnp.int32, L)

        def body(g, carry):
            idx0 = lanes * F + g * (L * F)
            acc = jnp.zeros((L,), jnp.float32)
            for f in range(F):
                acc = acc + plsc.load_gather(vals_v, [idx0 + f])
            out_v[pl.ds(g * L, L)] = acc
            return carry

        lax.fori_loop(0, b_per_w // L, body, 0)
        pltpu.sync_copy(out_v, out_hbm.at[pl.ds(base, b_per_w)])

    return lr_pool


def kernel(fids_batch, table):
    B, F = fids_batch.shape
    fids_flat = fids_batch.reshape(B * F)
    return _build(B, F)(fids_flat, table)
